# baseline (speedup 1.0000x reference)
import functools

import jax
import jax.numpy as jnp
from jax import lax
from jax.experimental import pallas as pl
from jax.experimental.pallas import tpu as pltpu
from jax.experimental.pallas import tpu_sc as plsc

N_NODES = 10000
N_EDGES = 160000
D_FEAT = 256
D_PACK = D_FEAT // 2

NC = 2
NS = 16
NW = NC * NS
LANES = 16
CHUNK = 192
NBUF = 2
NCHUNK = 27
PER_W = CHUNK * NCHUNK
E_PAD = PER_W * NW

_BITREV = [int("{:04b}".format(i)[::-1], 2) for i in range(LANES)]
_HI_MASK = -65536


def _rot(v, s, idx):
    dnums = lax.GatherDimensionNumbers(
        offset_dims=(), collapsed_slice_dims=(0,), start_index_map=(0,))
    return lax.gather(v, (idx ^ s)[:, None], dnums, (1,),
                      mode=lax.GatherScatterMode.PROMISE_IN_BOUNDS)


def _tree16(vs, idx):
    level = vs
    for s in (8, 4, 2, 1):
        nxt = []
        for i in range(0, len(level), 2):
            a, b = level[i], level[i + 1]
            nxt.append(jnp.where((idx & (2 * s - 1)) < s,
                                 a + _rot(a, s, idx), b + _rot(b, s, idx)))
        level = nxt
    return level[0]


def _make_sc_kernel():
    mesh = plsc.VectorSubcoreMesh(core_axis_name="c", subcore_axis_name="s")

    @functools.partial(
        pl.kernel,
        mesh=mesh,
        out_type=jax.ShapeDtypeStruct((E_PAD,), jnp.float32),
        scratch_types=[
            pltpu.VMEM((2 * PER_W,), jnp.int32),
        ]
        + [pltpu.VMEM((2 * CHUNK, D_PACK), jnp.int32)] * NBUF
        + [pltpu.VMEM((PER_W,), jnp.float32)]
        + [pltpu.SemaphoreType.DMA] * NBUF,
    )
    def k(tbl_hbm, idx_hbm, out_hbm, idx_v, *rest):
        buf = rest[0:NBUF]
        out_all = rest[NBUF]
        sem = rest[NBUF + 1:]

        wid = lax.axis_index("s") * NC + lax.axis_index("c")
        base = wid * PER_W
        pltpu.sync_copy(idx_hbm.at[pl.ds(2 * base, 2 * PER_W)], idx_v)

        idx = lax.iota(jnp.int32, LANES)

        def start(c, b):
            pltpu.async_copy(
                tbl_hbm.at[idx_v.at[pl.ds(c * 2 * CHUNK, 2 * CHUNK)]],
                buf[b], sem[b])

        for b0 in range(NBUF):
            start(b0, b0)

        def wait(b):
            pltpu.make_async_copy(
                tbl_hbm.at[pl.ds(0, 2 * CHUNK)], buf[b], sem[b]).wait()

        def edge_acc(rows, r):
            acc = None
            for w in range(D_PACK // LANES):
                uw = rows[r, pl.ds(w * LANES, LANES)]
                vw = rows[CHUNK + r, pl.ds(w * LANES, LANES)]
                u_lo = lax.bitcast_convert_type(uw << 16, jnp.float32)
                u_hi = lax.bitcast_convert_type(uw & _HI_MASK, jnp.float32)
                v_lo = lax.bitcast_convert_type(vw << 16, jnp.float32)
                v_hi = lax.bitcast_convert_type(vw & _HI_MASK, jnp.float32)
                term = u_lo * v_lo + u_hi * v_hi
                acc = term if acc is None else acc + term
            return acc

        def compute(c, b):
            def group_body(g, _):
                accs = []
                for p in range(LANES):
                    r = g * LANES + _BITREV[p]
                    accs.append(edge_acc(buf[b], r))
                out_all[pl.ds(c * CHUNK + g * LANES, LANES)] = (
                    _tree16(accs, idx))
                return 0

            lax.fori_loop(0, CHUNK // LANES, group_body, 0)

        def chunk_tuple(jj, _):
            for b in range(NBUF):
                c = NBUF * jj + b
                wait(b)
                compute(c, b)

                @pl.when(c + NBUF < NCHUNK)
                def _(b=b, c=c):
                    start(c + NBUF, b)
            return 0

        lax.fori_loop(0, NCHUNK // NBUF, chunk_tuple, 0)

        c_last = NCHUNK - 1
        b_last = c_last % NBUF
        wait(b_last)
        compute(c_last, b_last)
        pltpu.sync_copy(out_all, out_hbm.at[pl.ds(base, PER_W)])

    return k


_sc_kernel = _make_sc_kernel()


@jax.jit
def kernel(new_ft, raw_ft, edge_index):
    tbl = jnp.concatenate(
        [new_ft.astype(jnp.bfloat16), raw_ft.astype(jnp.bfloat16)])
    tbl = lax.bitcast_convert_type(
        tbl.reshape(2 * N_NODES, D_PACK, 2), jnp.int32)
    src = edge_index[0].astype(jnp.int32)
    dst = edge_index[1].astype(jnp.int32)
    pad = jnp.zeros((E_PAD - N_EDGES,), jnp.int32)
    src = jnp.concatenate([src, pad]).reshape(NW, NCHUNK, 1, CHUNK)
    dst = (jnp.concatenate([dst, pad]).reshape(NW, NCHUNK, 1, CHUNK)
           + N_NODES)
    comb = jnp.concatenate([src, dst], axis=2).reshape(-1)
    score = _sc_kernel(tbl, comb)
    return score[:N_EDGES].reshape(N_EDGES, 1)


# speedup vs baseline: 2.3296x; 2.3296x over previous
import functools

import jax
import jax.numpy as jnp
from jax import lax
from jax.experimental import pallas as pl
from jax.experimental.pallas import tpu as pltpu
from jax.experimental.pallas import tpu_sc as plsc

N_NODES = 10000
N_EDGES = 160000
D_FEAT = 256

NC = 2
NS = 16
NW = NC * NS
LANES = 16
CHUNK = 96
NBUF = 2
NCHUNK = 53
PER_W = CHUNK * NCHUNK
E_PAD = PER_W * NW

_BITREV = [int("{:04b}".format(i)[::-1], 2) for i in range(LANES)]
_HI_MASK = -65536


def _rot(v, s, idx):
    dnums = lax.GatherDimensionNumbers(
        offset_dims=(), collapsed_slice_dims=(0,), start_index_map=(0,))
    return lax.gather(v, (idx ^ s)[:, None], dnums, (1,),
                      mode=lax.GatherScatterMode.PROMISE_IN_BOUNDS)


def _tree16(vs, idx):
    level = vs
    for s in (8, 4, 2, 1):
        nxt = []
        for i in range(0, len(level), 2):
            a, b = level[i], level[i + 1]
            nxt.append(jnp.where((idx & (2 * s - 1)) < s,
                                 a + _rot(a, s, idx), b + _rot(b, s, idx)))
        level = nxt
    return level[0]


def _make_sc_kernel():
    mesh = plsc.VectorSubcoreMesh(core_axis_name="c", subcore_axis_name="s")

    @functools.partial(
        pl.kernel,
        mesh=mesh,
        out_type=jax.ShapeDtypeStruct((E_PAD,), jnp.float32),
        scratch_types=[
            pltpu.VMEM((PER_W,), jnp.int32),
            pltpu.VMEM((PER_W,), jnp.int32),
        ]
        + [pltpu.VMEM((CHUNK, D_FEAT), jnp.float32)] * (2 * NBUF)
        + [pltpu.VMEM((PER_W,), jnp.float32)]
        + [pltpu.SemaphoreType.DMA] * (2 * NBUF),
    )
    def k(new_hbm, raw_hbm, src_hbm, dst_hbm, out_hbm, src_v, dst_v, *rest):
        ubuf = rest[0:NBUF]
        vbuf = rest[NBUF:2 * NBUF]
        out_all = rest[2 * NBUF]
        usem = rest[2 * NBUF + 1:2 * NBUF + 1 + NBUF]
        vsem = rest[2 * NBUF + 1 + NBUF:]

        wid = lax.axis_index("s") * NC + lax.axis_index("c")
        base = wid * PER_W
        pltpu.sync_copy(src_hbm.at[pl.ds(base, PER_W)], src_v)
        pltpu.sync_copy(dst_hbm.at[pl.ds(base, PER_W)], dst_v)

        idx = lax.iota(jnp.int32, LANES)

        def start(c, b):
            off = c * CHUNK
            pltpu.async_copy(
                new_hbm.at[src_v.at[pl.ds(off, CHUNK)]], ubuf[b], usem[b])
            pltpu.async_copy(
                raw_hbm.at[dst_v.at[pl.ds(off, CHUNK)]], vbuf[b], vsem[b])

        for b0 in range(NBUF):
            start(b0, b0)

        def wait(b):
            pltpu.make_async_copy(
                new_hbm.at[pl.ds(0, CHUNK)], ubuf[b], usem[b]).wait()
            pltpu.make_async_copy(
                raw_hbm.at[pl.ds(0, CHUNK)], vbuf[b], vsem[b]).wait()

        def edge_acc(u_rows, v_rows, r):
            acc = None
            for w in range(D_FEAT // LANES):
                uw = u_rows[r, pl.ds(w * LANES, LANES)]
                vw = v_rows[r, pl.ds(w * LANES, LANES)]
                term = uw * vw
                acc = term if acc is None else acc + term
            return acc

        def compute(c, b):
            def group_body(g, _):
                accs = []
                for p in range(LANES):
                    r = g * LANES + _BITREV[p]
                    accs.append(edge_acc(ubuf[b], vbuf[b], r))
                out_all[pl.ds(c * CHUNK + g * LANES, LANES)] = (
                    _tree16(accs, idx))
                return 0

            lax.fori_loop(0, CHUNK // LANES, group_body, 0)

        def chunk_tuple(jj, _):
            for b in range(NBUF):
                c = NBUF * jj + b
                wait(b)
                compute(c, b)

                @pl.when(c + NBUF < NCHUNK)
                def _(b=b, c=c):
                    start(c + NBUF, b)
            return 0

        lax.fori_loop(0, NCHUNK // NBUF, chunk_tuple, 0)

        c_last = NCHUNK - 1
        b_last = c_last % NBUF
        wait(b_last)
        compute(c_last, b_last)
        pltpu.sync_copy(out_all, out_hbm.at[pl.ds(base, PER_W)])

    return k


_sc_kernel = _make_sc_kernel()


@jax.jit
def kernel(new_ft, raw_ft, edge_index):
    src = edge_index[0].astype(jnp.int32)
    dst = edge_index[1].astype(jnp.int32)
    pad = jnp.zeros((E_PAD - N_EDGES,), jnp.int32)
    src = jnp.concatenate([src, pad])
    dst = jnp.concatenate([dst, pad])
    score = _sc_kernel(new_ft, raw_ft, src, dst)
    return score[:N_EDGES].reshape(N_EDGES, 1)


# speedup vs baseline: 2.5171x; 1.0805x over previous
import functools

import jax
import jax.numpy as jnp
from jax import lax
from jax.experimental import pallas as pl
from jax.experimental.pallas import tpu as pltpu
from jax.experimental.pallas import tpu_sc as plsc

N_NODES = 10000
N_EDGES = 160000
D_FEAT = 256
D_PACK = D_FEAT // 2

NC = 2
NS = 16
NW = NC * NS
LANES = 16
CHUNK = 192
NBUF = 2
NCHUNK = 27
PER_W = CHUNK * NCHUNK
E_PAD = PER_W * NW

_BITREV = [int("{:04b}".format(i)[::-1], 2) for i in range(LANES)]
_HI_MASK = -65536


def _rot(v, s, idx):
    dnums = lax.GatherDimensionNumbers(
        offset_dims=(), collapsed_slice_dims=(0,), start_index_map=(0,))
    return lax.gather(v, (idx ^ s)[:, None], dnums, (1,),
                      mode=lax.GatherScatterMode.PROMISE_IN_BOUNDS)


def _tree16(vs, idx):
    level = vs
    for s in (8, 4, 2, 1):
        nxt = []
        for i in range(0, len(level), 2):
            a, b = level[i], level[i + 1]
            nxt.append(jnp.where((idx & (2 * s - 1)) < s,
                                 a + _rot(a, s, idx), b + _rot(b, s, idx)))
        level = nxt
    return level[0]


def _pack_bf16_pairs(x):
    u = lax.bitcast_convert_type(x, jnp.uint32)
    lo, hi = u[:, :D_PACK], u[:, D_PACK:]
    lo_r = (lo + jnp.uint32(0x7FFF) + ((lo >> 16) & jnp.uint32(1))) >> 16
    hi_r = ((hi + jnp.uint32(0x7FFF) + ((hi >> 16) & jnp.uint32(1)))
            & jnp.uint32(0xFFFF0000))
    return lax.bitcast_convert_type(lo_r | hi_r, jnp.int32)


def _make_sc_kernel():
    mesh = plsc.VectorSubcoreMesh(core_axis_name="c", subcore_axis_name="s")

    @functools.partial(
        pl.kernel,
        mesh=mesh,
        out_type=jax.ShapeDtypeStruct((E_PAD,), jnp.float32),
        scratch_types=[
            pltpu.VMEM((PER_W,), jnp.int32),
            pltpu.VMEM((PER_W,), jnp.int32),
        ]
        + [pltpu.VMEM((CHUNK, D_PACK), jnp.int32)] * (2 * NBUF)
        + [pltpu.VMEM((PER_W,), jnp.float32)]
        + [pltpu.SemaphoreType.DMA] * (2 * NBUF),
    )
    def k(new_hbm, raw_hbm, src_hbm, dst_hbm, out_hbm, src_v, dst_v, *rest):
        ubuf = rest[0:NBUF]
        vbuf = rest[NBUF:2 * NBUF]
        out_all = rest[2 * NBUF]
        usem = rest[2 * NBUF + 1:2 * NBUF + 1 + NBUF]
        vsem = rest[2 * NBUF + 1 + NBUF:]

        wid = lax.axis_index("s") * NC + lax.axis_index("c")
        base = wid * PER_W
        pltpu.sync_copy(src_hbm.at[pl.ds(base, PER_W)], src_v)
        pltpu.sync_copy(dst_hbm.at[pl.ds(base, PER_W)], dst_v)

        idx = lax.iota(jnp.int32, LANES)

        def start(c, b):
            off = c * CHUNK
            pltpu.async_copy(
                new_hbm.at[src_v.at[pl.ds(off, CHUNK)]], ubuf[b], usem[b])
            pltpu.async_copy(
                raw_hbm.at[dst_v.at[pl.ds(off, CHUNK)]], vbuf[b], vsem[b])

        for b0 in range(NBUF):
            start(b0, b0)

        def wait(b):
            pltpu.make_async_copy(
                new_hbm.at[pl.ds(0, CHUNK)], ubuf[b], usem[b]).wait()
            pltpu.make_async_copy(
                raw_hbm.at[pl.ds(0, CHUNK)], vbuf[b], vsem[b]).wait()

        def edge_acc(u_rows, v_rows, r):
            acc = None
            for w in range(D_PACK // LANES):
                uw = u_rows[r, pl.ds(w * LANES, LANES)]
                vw = v_rows[r, pl.ds(w * LANES, LANES)]
                u_lo = lax.bitcast_convert_type(uw << 16, jnp.float32)
                u_hi = lax.bitcast_convert_type(uw & _HI_MASK, jnp.float32)
                v_lo = lax.bitcast_convert_type(vw << 16, jnp.float32)
                v_hi = lax.bitcast_convert_type(vw & _HI_MASK, jnp.float32)
                term = u_lo * v_lo + u_hi * v_hi
                acc = term if acc is None else acc + term
            return acc

        def compute(c, b):
            def group_body(g, _):
                accs = []
                for p in range(LANES):
                    r = g * LANES + _BITREV[p]
                    accs.append(edge_acc(ubuf[b], vbuf[b], r))
                out_all[pl.ds(c * CHUNK + g * LANES, LANES)] = (
                    _tree16(accs, idx))
                return 0

            lax.fori_loop(0, CHUNK // LANES, group_body, 0)

        def chunk_tuple(jj, _):
            for b in range(NBUF):
                c = NBUF * jj + b
                wait(b)
                compute(c, b)

                @pl.when(c + NBUF < NCHUNK)
                def _(b=b, c=c):
                    start(c + NBUF, b)
            return 0

        lax.fori_loop(0, NCHUNK // NBUF, chunk_tuple, 0)

        c_last = NCHUNK - 1
        b_last = c_last % NBUF
        wait(b_last)
        compute(c_last, b_last)
        pltpu.sync_copy(out_all, out_hbm.at[pl.ds(base, PER_W)])

    return k


_sc_kernel = _make_sc_kernel()


@jax.jit
def kernel(new_ft, raw_ft, edge_index):
    new_p = _pack_bf16_pairs(new_ft)
    raw_p = _pack_bf16_pairs(raw_ft)
    src = edge_index[0].astype(jnp.int32)
    dst = edge_index[1].astype(jnp.int32)
    pad = jnp.zeros((E_PAD - N_EDGES,), jnp.int32)
    src = jnp.concatenate([src, pad])
    dst = jnp.concatenate([dst, pad])
    score = _sc_kernel(new_p, raw_p, src, dst)
    return score[:N_EDGES].reshape(N_EDGES, 1)


# speedup vs baseline: 2.6398x; 1.0487x over previous
import functools

import jax
import jax.numpy as jnp
from jax import lax
from jax.experimental import pallas as pl
from jax.experimental.pallas import tpu as pltpu
from jax.experimental.pallas import tpu_sc as plsc

N_NODES = 10000
N_EDGES = 160000
D_FEAT = 256
D_PACK = D_FEAT // 2

NC = 2
NS = 16
NW = NC * NS
LANES = 16
CHUNK = 128
NBUF = 3
NCHUNK = 40
PER_W = CHUNK * NCHUNK
E_PAD = PER_W * NW

_BITREV = [int("{:04b}".format(i)[::-1], 2) for i in range(LANES)]
_HI_MASK = -65536


def _rot(v, s, idx):
    dnums = lax.GatherDimensionNumbers(
        offset_dims=(), collapsed_slice_dims=(0,), start_index_map=(0,))
    return lax.gather(v, (idx ^ s)[:, None], dnums, (1,),
                      mode=lax.GatherScatterMode.PROMISE_IN_BOUNDS)


def _tree16(vs, idx):
    level = vs
    for s in (8, 4, 2, 1):
        nxt = []
        for i in range(0, len(level), 2):
            a, b = level[i], level[i + 1]
            nxt.append(jnp.where((idx & (2 * s - 1)) < s,
                                 a + _rot(a, s, idx), b + _rot(b, s, idx)))
        level = nxt
    return level[0]


def _pack_bf16_pairs(x):
    u = lax.bitcast_convert_type(x, jnp.uint32)
    lo, hi = u[:, :D_PACK], u[:, D_PACK:]
    lo_r = (lo + jnp.uint32(0x7FFF) + ((lo >> 16) & jnp.uint32(1))) >> 16
    hi_r = ((hi + jnp.uint32(0x7FFF) + ((hi >> 16) & jnp.uint32(1)))
            & jnp.uint32(0xFFFF0000))
    return lax.bitcast_convert_type(lo_r | hi_r, jnp.int32)


def _make_sc_kernel():
    mesh = plsc.VectorSubcoreMesh(core_axis_name="c", subcore_axis_name="s")

    @functools.partial(
        pl.kernel,
        mesh=mesh,
        out_type=jax.ShapeDtypeStruct((E_PAD,), jnp.float32),
        scratch_types=[
            pltpu.VMEM((PER_W,), jnp.int32),
            pltpu.VMEM((PER_W,), jnp.int32),
        ]
        + [pltpu.VMEM((CHUNK, D_PACK), jnp.int32)] * (2 * NBUF)
        + [pltpu.VMEM((PER_W,), jnp.float32)]
        + [pltpu.SemaphoreType.DMA] * (2 * NBUF),
    )
    def k(new_hbm, raw_hbm, src_hbm, dst_hbm, out_hbm, src_v, dst_v, *rest):
        ubuf = rest[0:NBUF]
        vbuf = rest[NBUF:2 * NBUF]
        out_all = rest[2 * NBUF]
        usem = rest[2 * NBUF + 1:2 * NBUF + 1 + NBUF]
        vsem = rest[2 * NBUF + 1 + NBUF:]

        wid = lax.axis_index("s") * NC + lax.axis_index("c")
        base = wid * PER_W
        pltpu.sync_copy(src_hbm.at[pl.ds(base, PER_W)], src_v)
        pltpu.sync_copy(dst_hbm.at[pl.ds(base, PER_W)], dst_v)

        idx = lax.iota(jnp.int32, LANES)

        def start(c, b):
            off = c * CHUNK
            pltpu.async_copy(
                new_hbm.at[src_v.at[pl.ds(off, CHUNK)]], ubuf[b], usem[b])
            pltpu.async_copy(
                raw_hbm.at[dst_v.at[pl.ds(off, CHUNK)]], vbuf[b], vsem[b])

        for b0 in range(NBUF):
            start(b0, b0)

        def wait(b):
            pltpu.make_async_copy(
                new_hbm.at[pl.ds(0, CHUNK)], ubuf[b], usem[b]).wait()
            pltpu.make_async_copy(
                raw_hbm.at[pl.ds(0, CHUNK)], vbuf[b], vsem[b]).wait()

        def edge_acc(u_rows, v_rows, r):
            acc = None
            for w in range(D_PACK // LANES):
                uw = u_rows[r, pl.ds(w * LANES, LANES)]
                vw = v_rows[r, pl.ds(w * LANES, LANES)]
                u_lo = lax.bitcast_convert_type(uw << 16, jnp.float32)
                u_hi = lax.bitcast_convert_type(uw & _HI_MASK, jnp.float32)
                v_lo = lax.bitcast_convert_type(vw << 16, jnp.float32)
                v_hi = lax.bitcast_convert_type(vw & _HI_MASK, jnp.float32)
                term = u_lo * v_lo + u_hi * v_hi
                acc = term if acc is None else acc + term
            return acc

        def compute(c, b):
            def group_body(g, _):
                accs = []
                for p in range(LANES):
                    r = g * LANES + _BITREV[p]
                    accs.append(edge_acc(ubuf[b], vbuf[b], r))
                out_all[pl.ds(c * CHUNK + g * LANES, LANES)] = (
                    _tree16(accs, idx))
                return 0

            lax.fori_loop(0, CHUNK // LANES, group_body, 0)

        def chunk_tuple(jj, _):
            for b in range(NBUF):
                c = NBUF * jj + b
                wait(b)
                compute(c, b)

                @pl.when(c + NBUF < NCHUNK)
                def _(b=b, c=c):
                    start(c + NBUF, b)
            return 0

        lax.fori_loop(0, NCHUNK // NBUF, chunk_tuple, 0)

        c_last = NCHUNK - 1
        b_last = c_last % NBUF
        wait(b_last)
        compute(c_last, b_last)
        pltpu.sync_copy(out_all, out_hbm.at[pl.ds(base, PER_W)])

    return k


_sc_kernel = _make_sc_kernel()


@jax.jit
def kernel(new_ft, raw_ft, edge_index):
    new_p = _pack_bf16_pairs(new_ft)
    raw_p = _pack_bf16_pairs(raw_ft)
    src = edge_index[0].astype(jnp.int32)
    dst = edge_index[1].astype(jnp.int32)
    pad = jnp.zeros((E_PAD - N_EDGES,), jnp.int32)
    src = jnp.concatenate([src, pad])
    dst = jnp.concatenate([dst, pad])
    score = _sc_kernel(new_p, raw_p, src, dst)
    return score[:N_EDGES].reshape(N_EDGES, 1)


# speedup vs baseline: 3.1721x; 1.2017x over previous
import functools

import jax
import jax.numpy as jnp
from jax import lax
from jax.experimental import pallas as pl
from jax.experimental.pallas import tpu as pltpu
from jax.experimental.pallas import tpu_sc as plsc

N_NODES = 10000
N_EDGES = 160000
D_FEAT = 256
D_PACK = D_FEAT // 2

NC = 2
NS = 16
NW = NC * NS
LANES = 16
CHUNK = 96
NBUF = 4
NCHUNK = 53
PER_W = CHUNK * NCHUNK
E_PAD = PER_W * NW

_BITREV = [int("{:04b}".format(i)[::-1], 2) for i in range(LANES)]
_HI_MASK = -65536


def _rot(v, s, idx):
    dnums = lax.GatherDimensionNumbers(
        offset_dims=(), collapsed_slice_dims=(0,), start_index_map=(0,))
    return lax.gather(v, (idx ^ s)[:, None], dnums, (1,),
                      mode=lax.GatherScatterMode.PROMISE_IN_BOUNDS)


def _tree16(vs, idx):
    level = vs
    for s in (8, 4, 2, 1):
        nxt = []
        for i in range(0, len(level), 2):
            a, b = level[i], level[i + 1]
            nxt.append(jnp.where((idx & (2 * s - 1)) < s,
                                 a + _rot(a, s, idx), b + _rot(b, s, idx)))
        level = nxt
    return level[0]


def _pack_bf16_pairs(x):
    u = lax.bitcast_convert_type(x, jnp.uint32)
    lo, hi = u[:, :D_PACK], u[:, D_PACK:]
    lo_r = (lo + jnp.uint32(0x7FFF) + ((lo >> 16) & jnp.uint32(1))) >> 16
    hi_r = ((hi + jnp.uint32(0x7FFF) + ((hi >> 16) & jnp.uint32(1)))
            & jnp.uint32(0xFFFF0000))
    return lax.bitcast_convert_type(lo_r | hi_r, jnp.int32)


def _make_sc_kernel():
    mesh = plsc.VectorSubcoreMesh(core_axis_name="c", subcore_axis_name="s")

    @functools.partial(
        pl.kernel,
        mesh=mesh,
        out_type=jax.ShapeDtypeStruct((E_PAD,), jnp.float32),
        scratch_types=[
            pltpu.VMEM((PER_W,), jnp.int32),
            pltpu.VMEM((PER_W,), jnp.int32),
        ]
        + [pltpu.VMEM((CHUNK, D_PACK), jnp.int32)] * (2 * NBUF)
        + [pltpu.VMEM((PER_W,), jnp.float32)]
        + [pltpu.SemaphoreType.DMA] * (2 * NBUF),
    )
    def k(new_hbm, raw_hbm, src_hbm, dst_hbm, out_hbm, src_v, dst_v, *rest):
        ubuf = rest[0:NBUF]
        vbuf = rest[NBUF:2 * NBUF]
        out_all = rest[2 * NBUF]
        usem = rest[2 * NBUF + 1:2 * NBUF + 1 + NBUF]
        vsem = rest[2 * NBUF + 1 + NBUF:]

        wid = lax.axis_index("s") * NC + lax.axis_index("c")
        base = wid * PER_W
        pltpu.sync_copy(src_hbm.at[pl.ds(base, PER_W)], src_v)
        pltpu.sync_copy(dst_hbm.at[pl.ds(base, PER_W)], dst_v)

        idx = lax.iota(jnp.int32, LANES)

        def start(c, b):
            off = c * CHUNK
            pltpu.async_copy(
                new_hbm.at[src_v.at[pl.ds(off, CHUNK)]], ubuf[b], usem[b])
            pltpu.async_copy(
                raw_hbm.at[dst_v.at[pl.ds(off, CHUNK)]], vbuf[b], vsem[b])

        for b0 in range(NBUF):
            start(b0, b0)

        def wait(b):
            pltpu.make_async_copy(
                new_hbm.at[pl.ds(0, CHUNK)], ubuf[b], usem[b]).wait()
            pltpu.make_async_copy(
                raw_hbm.at[pl.ds(0, CHUNK)], vbuf[b], vsem[b]).wait()

        def edge_acc(u_rows, v_rows, r):
            acc = None
            for w in range(D_PACK // LANES):
                uw = u_rows[r, pl.ds(w * LANES, LANES)]
                vw = v_rows[r, pl.ds(w * LANES, LANES)]
                u_lo = lax.bitcast_convert_type(uw << 16, jnp.float32)
                u_hi = lax.bitcast_convert_type(uw & _HI_MASK, jnp.float32)
                v_lo = lax.bitcast_convert_type(vw << 16, jnp.float32)
                v_hi = lax.bitcast_convert_type(vw & _HI_MASK, jnp.float32)
                term = u_lo * v_lo + u_hi * v_hi
                acc = term if acc is None else acc + term
            return acc

        def compute(c, b):
            def group_body(g, _):
                accs = []
                for p in range(LANES):
                    r = g * LANES + _BITREV[p]
                    accs.append(edge_acc(ubuf[b], vbuf[b], r))
                out_all[pl.ds(c * CHUNK + g * LANES, LANES)] = (
                    _tree16(accs, idx))
                return 0

            lax.fori_loop(0, CHUNK // LANES, group_body, 0)

        def chunk_tuple(jj, _):
            for b in range(NBUF):
                c = NBUF * jj + b
                wait(b)
                compute(c, b)

                @pl.when(c + NBUF < NCHUNK)
                def _(b=b, c=c):
                    start(c + NBUF, b)
            return 0

        lax.fori_loop(0, NCHUNK // NBUF, chunk_tuple, 0)

        c_last = NCHUNK - 1
        b_last = c_last % NBUF
        wait(b_last)
        compute(c_last, b_last)
        pltpu.sync_copy(out_all, out_hbm.at[pl.ds(base, PER_W)])

    return k


_sc_kernel = _make_sc_kernel()


@jax.jit
def kernel(new_ft, raw_ft, edge_index):
    new_p = _pack_bf16_pairs(new_ft)
    raw_p = _pack_bf16_pairs(raw_ft)
    src = edge_index[0].astype(jnp.int32)
    dst = edge_index[1].astype(jnp.int32)
    pad = jnp.zeros((E_PAD - N_EDGES,), jnp.int32)
    src = jnp.concatenate([src, pad])
    dst = jnp.concatenate([dst, pad])
    score = _sc_kernel(new_p, raw_p, src, dst)
    return score[:N_EDGES].reshape(N_EDGES, 1)


# speedup vs baseline: 3.9144x; 1.2340x over previous
import functools

import jax
import jax.numpy as jnp
from jax import lax
from jax.experimental import pallas as pl
from jax.experimental.pallas import tpu as pltpu
from jax.experimental.pallas import tpu_sc as plsc

N_NODES = 10000
N_EDGES = 160000
D_FEAT = 256
D_PACK = D_FEAT // 2

NC = 2
NS = 16
NW = NC * NS
LANES = 16
CHUNK = 64
NBUF = 6
NCHUNK = 79
PER_W = CHUNK * NCHUNK
E_PAD = PER_W * NW

_BITREV = [int("{:04b}".format(i)[::-1], 2) for i in range(LANES)]
_HI_MASK = -65536


def _rot(v, s, idx):
    dnums = lax.GatherDimensionNumbers(
        offset_dims=(), collapsed_slice_dims=(0,), start_index_map=(0,))
    return lax.gather(v, (idx ^ s)[:, None], dnums, (1,),
                      mode=lax.GatherScatterMode.PROMISE_IN_BOUNDS)


def _tree16(vs, idx):
    level = vs
    for s in (8, 4, 2, 1):
        nxt = []
        for i in range(0, len(level), 2):
            a, b = level[i], level[i + 1]
            nxt.append(jnp.where((idx & (2 * s - 1)) < s,
                                 a + _rot(a, s, idx), b + _rot(b, s, idx)))
        level = nxt
    return level[0]


def _pack_bf16_pairs(x):
    u = lax.bitcast_convert_type(x, jnp.uint32)
    lo, hi = u[:, :D_PACK], u[:, D_PACK:]
    lo_r = (lo + jnp.uint32(0x7FFF) + ((lo >> 16) & jnp.uint32(1))) >> 16
    hi_r = ((hi + jnp.uint32(0x7FFF) + ((hi >> 16) & jnp.uint32(1)))
            & jnp.uint32(0xFFFF0000))
    return lax.bitcast_convert_type(lo_r | hi_r, jnp.int32)


def _make_sc_kernel():
    mesh = plsc.VectorSubcoreMesh(core_axis_name="c", subcore_axis_name="s")

    @functools.partial(
        pl.kernel,
        mesh=mesh,
        out_type=jax.ShapeDtypeStruct((E_PAD,), jnp.float32),
        scratch_types=[
            pltpu.VMEM((PER_W,), jnp.int32),
            pltpu.VMEM((PER_W,), jnp.int32),
        ]
        + [pltpu.VMEM((CHUNK, D_PACK), jnp.int32)] * (2 * NBUF)
        + [pltpu.VMEM((PER_W,), jnp.float32)]
        + [pltpu.SemaphoreType.DMA] * (2 * NBUF),
    )
    def k(new_hbm, raw_hbm, src_hbm, dst_hbm, out_hbm, src_v, dst_v, *rest):
        ubuf = rest[0:NBUF]
        vbuf = rest[NBUF:2 * NBUF]
        out_all = rest[2 * NBUF]
        usem = rest[2 * NBUF + 1:2 * NBUF + 1 + NBUF]
        vsem = rest[2 * NBUF + 1 + NBUF:]

        wid = lax.axis_index("s") * NC + lax.axis_index("c")
        base = wid * PER_W
        pltpu.sync_copy(src_hbm.at[pl.ds(base, PER_W)], src_v)
        pltpu.sync_copy(dst_hbm.at[pl.ds(base, PER_W)], dst_v)

        idx = lax.iota(jnp.int32, LANES)

        def start(c, b):
            off = c * CHUNK
            pltpu.async_copy(
                new_hbm.at[src_v.at[pl.ds(off, CHUNK)]], ubuf[b], usem[b])
            pltpu.async_copy(
                raw_hbm.at[dst_v.at[pl.ds(off, CHUNK)]], vbuf[b], vsem[b])

        for b0 in range(NBUF):
            start(b0, b0)

        def wait(b):
            pltpu.make_async_copy(
                new_hbm.at[pl.ds(0, CHUNK)], ubuf[b], usem[b]).wait()
            pltpu.make_async_copy(
                raw_hbm.at[pl.ds(0, CHUNK)], vbuf[b], vsem[b]).wait()

        def edge_acc(u_rows, v_rows, r):
            acc = None
            for w in range(D_PACK // LANES):
                uw = u_rows[r, pl.ds(w * LANES, LANES)]
                vw = v_rows[r, pl.ds(w * LANES, LANES)]
                u_lo = lax.bitcast_convert_type(uw << 16, jnp.float32)
                u_hi = lax.bitcast_convert_type(uw & _HI_MASK, jnp.float32)
                v_lo = lax.bitcast_convert_type(vw << 16, jnp.float32)
                v_hi = lax.bitcast_convert_type(vw & _HI_MASK, jnp.float32)
                term = u_lo * v_lo + u_hi * v_hi
                acc = term if acc is None else acc + term
            return acc

        def compute(c, b):
            def group_body(g, _):
                accs = []
                for p in range(LANES):
                    r = g * LANES + _BITREV[p]
                    accs.append(edge_acc(ubuf[b], vbuf[b], r))
                out_all[pl.ds(c * CHUNK + g * LANES, LANES)] = (
                    _tree16(accs, idx))
                return 0

            lax.fori_loop(0, CHUNK // LANES, group_body, 0)

        def chunk_tuple(jj, _):
            for b in range(NBUF):
                c = NBUF * jj + b
                wait(b)
                compute(c, b)

                @pl.when(c + NBUF < NCHUNK)
                def _(b=b, c=c):
                    start(c + NBUF, b)
            return 0

        lax.fori_loop(0, NCHUNK // NBUF, chunk_tuple, 0)

        c_last = NCHUNK - 1
        b_last = c_last % NBUF
        wait(b_last)
        compute(c_last, b_last)
        pltpu.sync_copy(out_all, out_hbm.at[pl.ds(base, PER_W)])

    return k


_sc_kernel = _make_sc_kernel()


@jax.jit
def kernel(new_ft, raw_ft, edge_index):
    new_p = _pack_bf16_pairs(new_ft)
    raw_p = _pack_bf16_pairs(raw_ft)
    src = edge_index[0].astype(jnp.int32)
    dst = edge_index[1].astype(jnp.int32)
    pad = jnp.zeros((E_PAD - N_EDGES,), jnp.int32)
    src = jnp.concatenate([src, pad])
    dst = jnp.concatenate([dst, pad])
    score = _sc_kernel(new_p, raw_p, src, dst)
    return score[:N_EDGES].reshape(N_EDGES, 1)


# speedup vs baseline: 4.3255x; 1.1050x over previous
import functools

import jax
import jax.numpy as jnp
from jax import lax
from jax.experimental import pallas as pl
from jax.experimental.pallas import tpu as pltpu
from jax.experimental.pallas import tpu_sc as plsc

N_NODES = 10000
N_EDGES = 160000
D_FEAT = 256
D_PACK = D_FEAT // 2

NC = 2
NS = 16
NW = NC * NS
LANES = 16
CHUNK = 48
NBUF = 8
NCHUNK = 105
PER_W = CHUNK * NCHUNK
E_PAD = PER_W * NW

_BITREV = [int("{:04b}".format(i)[::-1], 2) for i in range(LANES)]
_HI_MASK = -65536


def _rot(v, s, idx):
    dnums = lax.GatherDimensionNumbers(
        offset_dims=(), collapsed_slice_dims=(0,), start_index_map=(0,))
    return lax.gather(v, (idx ^ s)[:, None], dnums, (1,),
                      mode=lax.GatherScatterMode.PROMISE_IN_BOUNDS)


def _tree16(vs, idx):
    level = vs
    for s in (8, 4, 2, 1):
        nxt = []
        for i in range(0, len(level), 2):
            a, b = level[i], level[i + 1]
            nxt.append(jnp.where((idx & (2 * s - 1)) < s,
                                 a + _rot(a, s, idx), b + _rot(b, s, idx)))
        level = nxt
    return level[0]


def _pack_bf16_pairs(x):
    u = lax.bitcast_convert_type(x, jnp.uint32)
    lo, hi = u[:, :D_PACK], u[:, D_PACK:]
    lo_r = (lo + jnp.uint32(0x7FFF) + ((lo >> 16) & jnp.uint32(1))) >> 16
    hi_r = ((hi + jnp.uint32(0x7FFF) + ((hi >> 16) & jnp.uint32(1)))
            & jnp.uint32(0xFFFF0000))
    return lax.bitcast_convert_type(lo_r | hi_r, jnp.int32)


def _make_sc_kernel():
    mesh = plsc.VectorSubcoreMesh(core_axis_name="c", subcore_axis_name="s")

    @functools.partial(
        pl.kernel,
        mesh=mesh,
        out_type=jax.ShapeDtypeStruct((E_PAD,), jnp.float32),
        scratch_types=[
            pltpu.VMEM((PER_W,), jnp.int32),
            pltpu.VMEM((PER_W,), jnp.int32),
        ]
        + [pltpu.VMEM((CHUNK, D_PACK), jnp.int32)] * (2 * NBUF)
        + [pltpu.VMEM((PER_W,), jnp.float32)]
        + [pltpu.SemaphoreType.DMA] * (2 * NBUF),
    )
    def k(new_hbm, raw_hbm, src_hbm, dst_hbm, out_hbm, src_v, dst_v, *rest):
        ubuf = rest[0:NBUF]
        vbuf = rest[NBUF:2 * NBUF]
        out_all = rest[2 * NBUF]
        usem = rest[2 * NBUF + 1:2 * NBUF + 1 + NBUF]
        vsem = rest[2 * NBUF + 1 + NBUF:]

        wid = lax.axis_index("s") * NC + lax.axis_index("c")
        base = wid * PER_W
        pltpu.sync_copy(src_hbm.at[pl.ds(base, PER_W)], src_v)
        pltpu.sync_copy(dst_hbm.at[pl.ds(base, PER_W)], dst_v)

        idx = lax.iota(jnp.int32, LANES)

        def start(c, b):
            off = c * CHUNK
            pltpu.async_copy(
                new_hbm.at[src_v.at[pl.ds(off, CHUNK)]], ubuf[b], usem[b])
            pltpu.async_copy(
                raw_hbm.at[dst_v.at[pl.ds(off, CHUNK)]], vbuf[b], vsem[b])

        for b0 in range(NBUF):
            start(b0, b0)

        def wait(b):
            pltpu.make_async_copy(
                new_hbm.at[pl.ds(0, CHUNK)], ubuf[b], usem[b]).wait()
            pltpu.make_async_copy(
                raw_hbm.at[pl.ds(0, CHUNK)], vbuf[b], vsem[b]).wait()

        def edge_acc(u_rows, v_rows, r):
            acc = None
            for w in range(D_PACK // LANES):
                uw = u_rows[r, pl.ds(w * LANES, LANES)]
                vw = v_rows[r, pl.ds(w * LANES, LANES)]
                u_lo = lax.bitcast_convert_type(uw << 16, jnp.float32)
                u_hi = lax.bitcast_convert_type(uw & _HI_MASK, jnp.float32)
                v_lo = lax.bitcast_convert_type(vw << 16, jnp.float32)
                v_hi = lax.bitcast_convert_type(vw & _HI_MASK, jnp.float32)
                term = u_lo * v_lo + u_hi * v_hi
                acc = term if acc is None else acc + term
            return acc

        def compute(c, b):
            def group_body(g, _):
                accs = []
                for p in range(LANES):
                    r = g * LANES + _BITREV[p]
                    accs.append(edge_acc(ubuf[b], vbuf[b], r))
                out_all[pl.ds(c * CHUNK + g * LANES, LANES)] = (
                    _tree16(accs, idx))
                return 0

            lax.fori_loop(0, CHUNK // LANES, group_body, 0)

        def chunk_tuple(jj, _):
            for b in range(NBUF):
                c = NBUF * jj + b
                wait(b)
                compute(c, b)

                @pl.when(c + NBUF < NCHUNK)
                def _(b=b, c=c):
                    start(c + NBUF, b)
            return 0

        lax.fori_loop(0, NCHUNK // NBUF, chunk_tuple, 0)

        c_last = NCHUNK - 1
        b_last = c_last % NBUF
        wait(b_last)
        compute(c_last, b_last)
        pltpu.sync_copy(out_all, out_hbm.at[pl.ds(base, PER_W)])

    return k


_sc_kernel = _make_sc_kernel()


@jax.jit
def kernel(new_ft, raw_ft, edge_index):
    new_p = _pack_bf16_pairs(new_ft)
    raw_p = _pack_bf16_pairs(raw_ft)
    src = edge_index[0].astype(jnp.int32)
    dst = edge_index[1].astype(jnp.int32)
    pad = jnp.zeros((E_PAD - N_EDGES,), jnp.int32)
    src = jnp.concatenate([src, pad])
    dst = jnp.concatenate([dst, pad])
    score = _sc_kernel(new_p, raw_p, src, dst)
    return score[:N_EDGES].reshape(N_EDGES, 1)


# speedup vs baseline: 4.6409x; 1.0729x over previous
import functools

import jax
import jax.numpy as jnp
from jax import lax
from jax.experimental import pallas as pl
from jax.experimental.pallas import tpu as pltpu
from jax.experimental.pallas import tpu_sc as plsc

N_NODES = 10000
N_EDGES = 160000
D_FEAT = 256
D_PACK = D_FEAT // 2

NC = 2
NS = 16
NW = NC * NS
LANES = 16
CHUNK = 32
NBUF = 12
NCHUNK = 157
PER_W = CHUNK * NCHUNK
E_PAD = PER_W * NW

_BITREV = [int("{:04b}".format(i)[::-1], 2) for i in range(LANES)]
_HI_MASK = -65536


def _rot(v, s, idx):
    dnums = lax.GatherDimensionNumbers(
        offset_dims=(), collapsed_slice_dims=(0,), start_index_map=(0,))
    return lax.gather(v, (idx ^ s)[:, None], dnums, (1,),
                      mode=lax.GatherScatterMode.PROMISE_IN_BOUNDS)


def _tree16(vs, idx):
    level = vs
    for s in (8, 4, 2, 1):
        nxt = []
        for i in range(0, len(level), 2):
            a, b = level[i], level[i + 1]
            nxt.append(jnp.where((idx & (2 * s - 1)) < s,
                                 a + _rot(a, s, idx), b + _rot(b, s, idx)))
        level = nxt
    return level[0]


def _pack_bf16_pairs(x):
    u = lax.bitcast_convert_type(x, jnp.uint32)
    lo, hi = u[:, :D_PACK], u[:, D_PACK:]
    lo_r = (lo + jnp.uint32(0x7FFF) + ((lo >> 16) & jnp.uint32(1))) >> 16
    hi_r = ((hi + jnp.uint32(0x7FFF) + ((hi >> 16) & jnp.uint32(1)))
            & jnp.uint32(0xFFFF0000))
    return lax.bitcast_convert_type(lo_r | hi_r, jnp.int32)


def _make_sc_kernel():
    mesh = plsc.VectorSubcoreMesh(core_axis_name="c", subcore_axis_name="s")

    @functools.partial(
        pl.kernel,
        mesh=mesh,
        out_type=jax.ShapeDtypeStruct((E_PAD,), jnp.float32),
        scratch_types=[
            pltpu.VMEM((PER_W,), jnp.int32),
            pltpu.VMEM((PER_W,), jnp.int32),
        ]
        + [pltpu.VMEM((CHUNK, D_PACK), jnp.int32)] * (2 * NBUF)
        + [pltpu.VMEM((PER_W,), jnp.float32)]
        + [pltpu.SemaphoreType.DMA] * (2 * NBUF),
    )
    def k(new_hbm, raw_hbm, src_hbm, dst_hbm, out_hbm, src_v, dst_v, *rest):
        ubuf = rest[0:NBUF]
        vbuf = rest[NBUF:2 * NBUF]
        out_all = rest[2 * NBUF]
        usem = rest[2 * NBUF + 1:2 * NBUF + 1 + NBUF]
        vsem = rest[2 * NBUF + 1 + NBUF:]

        wid = lax.axis_index("s") * NC + lax.axis_index("c")
        base = wid * PER_W
        pltpu.sync_copy(src_hbm.at[pl.ds(base, PER_W)], src_v)
        pltpu.sync_copy(dst_hbm.at[pl.ds(base, PER_W)], dst_v)

        idx = lax.iota(jnp.int32, LANES)

        def start(c, b):
            off = c * CHUNK
            pltpu.async_copy(
                new_hbm.at[src_v.at[pl.ds(off, CHUNK)]], ubuf[b], usem[b])
            pltpu.async_copy(
                raw_hbm.at[dst_v.at[pl.ds(off, CHUNK)]], vbuf[b], vsem[b])

        for b0 in range(NBUF):
            start(b0, b0)

        def wait(b):
            pltpu.make_async_copy(
                new_hbm.at[pl.ds(0, CHUNK)], ubuf[b], usem[b]).wait()
            pltpu.make_async_copy(
                raw_hbm.at[pl.ds(0, CHUNK)], vbuf[b], vsem[b]).wait()

        def edge_acc(u_rows, v_rows, r):
            acc = None
            for w in range(D_PACK // LANES):
                uw = u_rows[r, pl.ds(w * LANES, LANES)]
                vw = v_rows[r, pl.ds(w * LANES, LANES)]
                u_lo = lax.bitcast_convert_type(uw << 16, jnp.float32)
                u_hi = lax.bitcast_convert_type(uw & _HI_MASK, jnp.float32)
                v_lo = lax.bitcast_convert_type(vw << 16, jnp.float32)
                v_hi = lax.bitcast_convert_type(vw & _HI_MASK, jnp.float32)
                term = u_lo * v_lo + u_hi * v_hi
                acc = term if acc is None else acc + term
            return acc

        def compute(c, b):
            def group_body(g, _):
                accs = []
                for p in range(LANES):
                    r = g * LANES + _BITREV[p]
                    accs.append(edge_acc(ubuf[b], vbuf[b], r))
                out_all[pl.ds(c * CHUNK + g * LANES, LANES)] = (
                    _tree16(accs, idx))
                return 0

            lax.fori_loop(0, CHUNK // LANES, group_body, 0)

        def chunk_tuple(jj, _):
            for b in range(NBUF):
                c = NBUF * jj + b
                wait(b)
                compute(c, b)

                @pl.when(c + NBUF < NCHUNK)
                def _(b=b, c=c):
                    start(c + NBUF, b)
            return 0

        lax.fori_loop(0, NCHUNK // NBUF, chunk_tuple, 0)

        c_last = NCHUNK - 1
        b_last = c_last % NBUF
        wait(b_last)
        compute(c_last, b_last)
        pltpu.sync_copy(out_all, out_hbm.at[pl.ds(base, PER_W)])

    return k


_sc_kernel = _make_sc_kernel()


@jax.jit
def kernel(new_ft, raw_ft, edge_index):
    new_p = _pack_bf16_pairs(new_ft)
    raw_p = _pack_bf16_pairs(raw_ft)
    src = edge_index[0].astype(jnp.int32)
    dst = edge_index[1].astype(jnp.int32)
    pad = jnp.zeros((E_PAD - N_EDGES,), jnp.int32)
    src = jnp.concatenate([src, pad])
    dst = jnp.concatenate([dst, pad])
    score = _sc_kernel(new_p, raw_p, src, dst)
    return score[:N_EDGES].reshape(N_EDGES, 1)
